# Initial kernel scaffold; baseline (speedup 1.0000x reference)
import jax
import jax.numpy as jnp
from jax.experimental import pallas as pl


def kernel(visit_segments, embedding_weight):
    raise NotImplementedError("write your pallas kernel here")


# speedup vs baseline: 5.1028x; 5.1028x over previous
import functools

import jax
import jax.numpy as jnp
from jax import lax
from jax.experimental import pallas as pl
from jax.experimental.pallas import tpu as pltpu
from jax.experimental.pallas import tpu_sc as plsc

R, S, D = 16384, 200, 32
B = R * S
IDX_MINOR = 128
NROWS = B // IDX_MINOR
NW = 32
CHUNK = 1024
K = CHUNK // IDX_MINOR
ROWS_PW = NROWS // NW
NCHUNK = ROWS_PW // K

_mesh = plsc.VectorSubcoreMesh(core_axis_name="c", subcore_axis_name="s")


@functools.partial(
    pl.kernel,
    mesh=_mesh,
    out_type=jax.ShapeDtypeStruct((B, D), jnp.float32),
    scratch_types=[
        pltpu.VMEM((K, IDX_MINOR), jnp.int32),
        pltpu.VMEM((CHUNK, D), jnp.float32),
        pltpu.SemaphoreType.DMA,
    ],
    compiler_params=pltpu.CompilerParams(use_tc_tiling_on_sc=False),
)
def _sc_gather(table_hbm, idx_hbm, out_hbm, idx_v, rows_v, gsem):
    wid = lax.axis_index("s") * 2 + lax.axis_index("c")
    row0 = wid * ROWS_PW

    def body(c, _):
        base_row = row0 + c * K
        pltpu.sync_copy(idx_hbm.at[pl.ds(base_row, K)], idx_v)
        copies = [
            pltpu.async_copy(
                table_hbm.at[idx_v.at[j]],
                rows_v.at[pl.ds(j * IDX_MINOR, IDX_MINOR)],
                gsem,
            )
            for j in range(K)
        ]
        for cp in copies:
            cp.wait()
        pltpu.sync_copy(rows_v, out_hbm.at[pl.ds(base_row * IDX_MINOR, CHUNK)])
        return ()

    lax.fori_loop(0, NCHUNK, body, (), unroll=False)


def kernel(visit_segments, embedding_weight):
    idx = visit_segments.reshape(NROWS, IDX_MINOR).astype(jnp.int32)
    out = _sc_gather(embedding_weight, idx)
    return out.reshape(R, S, D)
